# baseline (speedup 1.0000x reference)
import functools

import jax
import jax.numpy as jnp
from jax import lax
from jax.experimental import pallas as pl
from jax.experimental.pallas import tpu as pltpu
from jax.experimental.pallas import tpu_sc as plsc

_BATCH = 16384
_D = 32
_NCLASS = 100000
_NC = 2
_NS = 16
_L = 16
_NW = _NC * _NS
_BPW = _BATCH // _NW
_CH = 128

_mesh = plsc.VectorSubcoreMesh(core_axis_name="c", subcore_axis_name="s")


@functools.partial(
    pl.kernel,
    out_type=jax.ShapeDtypeStruct((_NW * _L,), jnp.float32),
    mesh=_mesh,
    compiler_params=pltpu.CompilerParams(
        needs_layout_passes=False, use_tc_tiling_on_sc=True,
        disable_bounds_checks=True, disable_semaphore_checks=True,
        skip_device_barrier=True),
    scratch_types=[
        pltpu.VMEM((_BPW,), jnp.int32),
        pltpu.VMEM((_CH, _D), jnp.float32),
        pltpu.VMEM((_CH, _D), jnp.float32),
        pltpu.VMEM((_L * _L,), jnp.float32),
        pltpu.VMEM((_L,), jnp.float32),
        pltpu.SemaphoreType.DMA,
        pltpu.SemaphoreType.DMA,
    ],
)
def _center_loss_sc(x_hbm, labels_hbm, centers_hbm, out_hbm,
                    idx_v, xv, cv, tmp, accv, semg, semx):
    wid = lax.axis_index("s") * _NC + lax.axis_index("c")
    base = wid * _BPW

    pltpu.sync_copy(labels_hbm.at[pl.ds(base, _BPW)], idx_v)

    lanes = lax.iota(jnp.int32, _L)

    def issue(g, carry):
        vec = idx_v[pl.ds(g * _L, _L)]
        for j in range(_L):
            r = vec[j]
            pltpu.async_copy(centers_hbm.at[pl.ds(r, 1)],
                             cv.at[pl.ds(g * _L + j - carry, 1)], semg)
        return carry

    def block(blk, acc):
        row0 = blk * _L
        for j in range(_L):
            d1 = xv[row0 + j, pl.ds(0, _L)] - cv[row0 + j, pl.ds(0, _L)]
            d2 = xv[row0 + j, pl.ds(_L, _L)] - cv[row0 + j, pl.ds(_L, _L)]
            tmp[pl.ds(j * _L, _L)] = d1 * d1 + d2 * d2
        dacc = jnp.zeros((_L,), jnp.float32)
        base_idx = lanes * _L
        for k in range(_L):
            dacc = dacc + plsc.load_gather(tmp, [base_idx + k])
        dist = jnp.clip(dacc, 1e-12, 1e12)
        return acc + dist

    acc = jnp.zeros((_L,), jnp.float32)
    for c in range(0, _BPW, _CH):
        xcp = pltpu.async_copy(x_hbm.at[pl.ds(base + c, _CH)], xv, semx)
        lax.fori_loop(c // _L, (c + _CH) // _L, issue, c)
        pltpu.make_async_copy(centers_hbm.at[pl.ds(0, _CH)], cv, semg).wait()
        xcp.wait()
        acc = lax.fori_loop(0, _CH // _L, block, acc)

    accv[...] = acc
    pltpu.sync_copy(accv, out_hbm.at[pl.ds(wid * _L, _L)])


def kernel(x, labels, centers):
    partials = _center_loss_sc(x, labels.astype(jnp.int32), centers)
    return jnp.sum(partials) * (1.0 / _BATCH)


# speedup vs baseline: 1.3734x; 1.3734x over previous
import functools

import jax
import jax.numpy as jnp
from jax import lax
from jax.experimental import pallas as pl
from jax.experimental.pallas import tpu as pltpu
from jax.experimental.pallas import tpu_sc as plsc

_BATCH = 16384
_D = 32
_NCLASS = 100000
_NC = 2
_NS = 16
_L = 16
_CH = 2048
_SPAN = _BATCH // _NS

_mesh = plsc.VectorSubcoreMesh(core_axis_name="c", subcore_axis_name="s")


@functools.partial(
    pl.kernel,
    out_type=jax.ShapeDtypeStruct((_NC, _BATCH), jnp.float32),
    mesh=_mesh,
    compiler_params=pltpu.CompilerParams(
        needs_layout_passes=False, use_tc_tiling_on_sc=True,
        disable_bounds_checks=True, disable_semaphore_checks=True,
        skip_device_barrier=True),
    scratch_types=[
        pltpu.VMEM((_NCLASS,), jnp.float32),
        pltpu.VMEM((_CH,), jnp.int32),
        pltpu.VMEM((_CH,), jnp.float32),
        pltpu.VMEM((_CH,), jnp.float32),
        pltpu.VMEM((_SPAN,), jnp.float32),
        pltpu.VMEM_SHARED((_NS, _BATCH), jnp.float32),
        pltpu.SemaphoreType.DMA,
        pltpu.SemaphoreType.DMA,
    ],
)
def _center_loss_sc(xt_hbm, labels_hbm, ct_hbm, out_hbm,
                    crow, labv, xrow, sqv, rbuf, stage, semc, semx):
    cid = lax.axis_index("c")
    sid = lax.axis_index("s")
    k = sid * _NC + cid

    ccp = pltpu.async_copy(ct_hbm.at[k], crow, semc)

    def group(g, carry):
        lv = labv[pl.ds(g * _L, _L)]
        cvals = plsc.load_gather(crow, [lv])
        xvals = xrow[pl.ds(g * _L, _L)]
        d = xvals - cvals
        sqv[pl.ds(g * _L, _L)] = d * d
        return carry

    for c0 in range(0, _BATCH, _CH):
        lcp = pltpu.async_copy(labels_hbm.at[pl.ds(c0, _CH)], labv, semx)
        xcp = pltpu.async_copy(xt_hbm.at[k, pl.ds(c0, _CH)], xrow, semx)
        if c0 == 0:
            ccp.wait()
        lcp.wait()
        xcp.wait()
        lax.fori_loop(0, _CH // _L, group, 0)
        pltpu.sync_copy(sqv, stage.at[sid, pl.ds(c0, _CH)])

    plsc.subcore_barrier()

    b0 = sid * _SPAN
    pltpu.sync_copy(stage.at[0, pl.ds(b0, _SPAN)], sqv.at[pl.ds(0, _SPAN)])
    for r in range(1, _NS):
        pltpu.sync_copy(stage.at[r, pl.ds(b0, _SPAN)], rbuf)
        for v in range(_SPAN // _L):
            s = pl.ds(v * _L, _L)
            sqv[s] = sqv[s] + rbuf[s]
    pltpu.sync_copy(sqv.at[pl.ds(0, _SPAN)],
                    out_hbm.at[cid, pl.ds(b0, _SPAN)])


def kernel(x, labels, centers):
    partials = _center_loss_sc(x.T, labels.astype(jnp.int32), centers.T)
    dist = partials[0] + partials[1]
    return jnp.mean(jnp.clip(dist, 1e-12, 1e12))


# speedup vs baseline: 1.4913x; 1.0858x over previous
import functools

import jax
import jax.numpy as jnp
from jax import lax
from jax.experimental import pallas as pl
from jax.experimental.pallas import tpu as pltpu
from jax.experimental.pallas import tpu_sc as plsc

_BATCH = 16384
_D = 32
_NCLASS = 100000
_NC = 2
_NS = 16
_L = 16
_CH = 2048
_SPAN = _BATCH // _NS

_mesh = plsc.VectorSubcoreMesh(core_axis_name="c", subcore_axis_name="s")


@functools.partial(
    pl.kernel,
    out_type=jax.ShapeDtypeStruct((_NC, _BATCH), jnp.float32),
    mesh=_mesh,
    compiler_params=pltpu.CompilerParams(
        needs_layout_passes=False, use_tc_tiling_on_sc=True,
        disable_bounds_checks=True, disable_semaphore_checks=True,
        skip_device_barrier=True),
    scratch_types=[
        pltpu.VMEM((_NCLASS,), jnp.float32),
        pltpu.VMEM((2, _CH), jnp.int32),
        pltpu.VMEM((2, _CH), jnp.float32),
        pltpu.VMEM((2, _CH), jnp.float32),
        pltpu.VMEM((_SPAN,), jnp.float32),
        pltpu.VMEM_SHARED((_NS, _BATCH), jnp.float32),
        pltpu.SemaphoreType.DMA,
        pltpu.SemaphoreType.DMA,
        pltpu.SemaphoreType.DMA,
        pltpu.SemaphoreType.DMA,
    ],
)
def _center_loss_sc(xt_hbm, labels_hbm, ct_hbm, out_hbm,
                    crow, labv, xrow, sqv, rbuf, stage,
                    semc, semx, sems0, sems1):
    cid = lax.axis_index("c")
    sid = lax.axis_index("s")
    k = sid * _NC + cid

    ccp = pltpu.async_copy(ct_hbm.at[k], crow, semc)

    nch = _BATCH // _CH

    def fetch(ci):
        c0 = ci * _CH
        b = ci % 2
        lcp = pltpu.async_copy(labels_hbm.at[pl.ds(c0, _CH)], labv.at[b],
                               semx)
        xcp = pltpu.async_copy(xt_hbm.at[k, pl.ds(c0, _CH)], xrow.at[b],
                               semx)
        return lcp, xcp

    def make_group(b):
        def group(g, carry):
            lv = labv[b, pl.ds(g * _L, _L)]
            cvals = plsc.load_gather(crow, [lv])
            xvals = xrow[b, pl.ds(g * _L, _L)]
            d = xvals - cvals
            sqv[b, pl.ds(g * _L, _L)] = d * d
            return carry
        return group

    sems = (sems0, sems1)
    pend = fetch(0)
    ccp.wait()
    for ci in range(nch):
        b = ci % 2
        nxt = fetch(ci + 1) if ci + 1 < nch else None
        for cp in pend:
            cp.wait()
        if ci >= 2:
            pltpu.make_async_copy(sqv.at[b], stage.at[sid, pl.ds(0, _CH)],
                                  sems[b]).wait()
        lax.fori_loop(0, _CH // _L, make_group(b), 0)
        pltpu.async_copy(sqv.at[b], stage.at[sid, pl.ds(ci * _CH, _CH)],
                         sems[b])
        pend = nxt
    for ci in (nch - 2, nch - 1):
        pltpu.make_async_copy(sqv.at[ci % 2],
                              stage.at[sid, pl.ds(0, _CH)],
                              sems[ci % 2]).wait()

    plsc.subcore_barrier()

    b0 = sid * _SPAN
    pltpu.sync_copy(stage.at[0, pl.ds(b0, _SPAN)],
                    sqv.at[0, pl.ds(0, _SPAN)])
    for r in range(1, _NS):
        pltpu.sync_copy(stage.at[r, pl.ds(b0, _SPAN)], rbuf)
        for v in range(_SPAN // _L):
            s = pl.ds(v * _L, _L)
            sqv[0, s] = sqv[0, s] + rbuf[s]
    pltpu.sync_copy(sqv.at[0, pl.ds(0, _SPAN)],
                    out_hbm.at[cid, pl.ds(b0, _SPAN)])


def kernel(x, labels, centers):
    partials = _center_loss_sc(x.T, labels.astype(jnp.int32), centers.T)
    dist = partials[0] + partials[1]
    return jnp.mean(jnp.clip(dist, 1e-12, 1e12))
